# baseline (speedup 1.0000x reference)
import functools

import jax
import jax.numpy as jnp
from jax import lax
from jax.experimental import pallas as pl
from jax.experimental.pallas import tpu as pltpu
from jax.experimental.pallas import tpu_sc as plsc

T = 4096
D = 1024
E = 8
I = 512
CAP = 1024
NSLOT = E * CAP
ZROW = NSLOT
NROW = 8320
DUMP = NROW
DISP_ROWS = 8448
NB = 8
BT = T // NB
NW = 32
TOK_W = T // NW
DCH = 32
CCH = 16


def _router_body(x_ref, wg_ref, tri_ref, d0_ref, d1_ref, ci0_ref, ci1_ref,
                 wk0_ref, wk1_ref, laux_ref,
                 ti0_s, ti1_s, p0_s, p1_s, w0_s, w1_s, off0, off1, me_acc):
    p = pl.program_id(0)
    b = pl.program_id(1)

    @pl.when(jnp.logical_and(p == 0, b == 0))
    def _init():
        off0[...] = jnp.zeros_like(off0)
        off1[...] = jnp.zeros_like(off1)
        me_acc[...] = jnp.zeros_like(me_acc)

    tok_sl = pl.ds(b * BT, BT)

    @pl.when(p == 0)
    def _pass0():
        xb = x_ref[...]
        wg = wg_ref[...]
        logits = jnp.dot(xb, wg, preferred_element_type=jnp.float32)
        m = jnp.max(logits, axis=1, keepdims=True)
        ex = jnp.exp(logits - m)
        gates = ex / jnp.sum(ex, axis=1, keepdims=True)

        lane = lax.broadcasted_iota(jnp.int32, (BT, E), 1)
        v0 = jnp.max(gates, axis=1, keepdims=True)
        i0 = jnp.min(jnp.where(gates == v0, lane, E), axis=1, keepdims=True)
        g1 = jnp.where(lane == i0, -jnp.inf, gates)
        v1 = jnp.max(g1, axis=1, keepdims=True)
        i1 = jnp.min(jnp.where(g1 == v1, lane, E), axis=1, keepdims=True)
        denom = v0 + v1 + 1e-9
        mask0 = (lane == i0).astype(jnp.float32)
        mask1 = (lane == i1).astype(jnp.float32)

        m01 = jnp.concatenate([mask0, mask1], axis=1)
        c01 = jnp.dot(tri_ref[...], m01, preferred_element_type=jnp.float32)
        c0 = c01[:, :E]
        c1 = c01[:, E:]

        o0 = off0[...]
        o1 = off1[...]
        pos0 = jnp.sum(mask0 * (c0 - 1.0 + o0), axis=1, keepdims=True)
        pos1 = jnp.sum(mask1 * (c1 - 1.0 + o1), axis=1, keepdims=True)
        off0[...] = o0 + jnp.sum(mask0, axis=0, keepdims=True)
        off1[...] = o1 + jnp.sum(mask1, axis=0, keepdims=True)
        me_acc[...] = me_acc[...] + jnp.sum(gates, axis=0, keepdims=True)

        ti0_s[tok_sl, :] = i0
        ti1_s[tok_sl, :] = i1
        p0_s[tok_sl, :] = pos0.astype(jnp.int32)
        p1_s[tok_sl, :] = pos1.astype(jnp.int32)
        w0_s[tok_sl, :] = v0 / denom
        w1_s[tok_sl, :] = v1 / denom

    @pl.when(p == 1)
    def _pass1():
        i0 = ti0_s[tok_sl, :]
        i1 = ti1_s[tok_sl, :]
        pos0 = p0_s[tok_sl, :]
        w0 = w0_s[tok_sl, :]
        w1 = w1_s[tok_sl, :]
        lane = lax.broadcasted_iota(jnp.int32, (BT, E), 1)
        mask1 = (lane == i1).astype(jnp.float32)
        cnt_g = jnp.sum(mask1 * off0[...], axis=1, keepdims=True)
        pos1 = p1_s[tok_sl, :] + cnt_g.astype(jnp.int32)

        k0 = pos0 < CAP
        k1 = pos1 < CAP
        s0 = i0 * CAP + pos0
        s1 = i1 * CAP + pos1
        d0 = jnp.where(k0, s0, DUMP)
        d1 = jnp.where(k1, s1, DUMP)
        d0_ref[...] = d0.reshape(4, 4, DCH)
        d1_ref[...] = d1.reshape(4, 4, DCH)
        ci0_ref[...] = jnp.where(k0, s0, ZROW).reshape(1, BT, 1)
        ci1_ref[...] = jnp.where(k1, s1, ZROW).reshape(1, BT, 1)
        wk0_ref[...] = jnp.where(k0, w0, 0.0).reshape(1, BT, 1)
        wk1_ref[...] = jnp.where(k1, w1, 0.0).reshape(1, BT, 1)

        @pl.when(b == NB - 1)
        def _fin():
            me = me_acc[...] / float(T)
            ce = off0[...] / float(T)
            laux_ref[...] = jnp.sum(me * ce).reshape(1, 1) * float(E)


def _router(x, wg):
    out_shapes = (
        jax.ShapeDtypeStruct((NW, 4, DCH), jnp.int32),
        jax.ShapeDtypeStruct((NW, 4, DCH), jnp.int32),
        jax.ShapeDtypeStruct((NB, BT, 1), jnp.int32),
        jax.ShapeDtypeStruct((NB, BT, 1), jnp.int32),
        jax.ShapeDtypeStruct((NB, BT, 1), jnp.float32),
        jax.ShapeDtypeStruct((NB, BT, 1), jnp.float32),
        jax.ShapeDtypeStruct((1, 1), jnp.float32),
    )
    blk = pl.BlockSpec((1, BT, 1), lambda p, b: (b, 0, 0))
    dblk = pl.BlockSpec((4, 4, DCH), lambda p, b: (b, 0, 0))
    return pl.pallas_call(
        _router_body,
        grid=(2, NB),
        in_specs=[
            pl.BlockSpec((BT, D), lambda p, b: (b * (1 - p), 0)),
            pl.BlockSpec((D, E), lambda p, b: (0, 0)),
            pl.BlockSpec((BT, BT), lambda p, b: (0, 0)),
        ],
        out_specs=(dblk, dblk, blk, blk, blk, blk,
                   pl.BlockSpec((1, 1), lambda p, b: (0, 0))),
        out_shape=out_shapes,
        scratch_shapes=[
            pltpu.VMEM((T, 1), jnp.int32),
            pltpu.VMEM((T, 1), jnp.int32),
            pltpu.VMEM((T, 1), jnp.int32),
            pltpu.VMEM((T, 1), jnp.int32),
            pltpu.VMEM((T, 1), jnp.float32),
            pltpu.VMEM((T, 1), jnp.float32),
            pltpu.VMEM((1, E), jnp.float32),
            pltpu.VMEM((1, E), jnp.float32),
            pltpu.VMEM((1, E), jnp.float32),
        ],
    )(x, wg, jnp.tril(jnp.ones((BT, BT), jnp.float32)))


def _dispatch_body(x_hbm, d0_hbm, d1_hbm, disp_hbm,
                   d0_v, d1_v, xb0_v, xb1_v, z_v, sem0, sem1):
    wid = lax.axis_index("s") * 2 + lax.axis_index("c")
    tbase = wid * TOK_W

    pltpu.sync_copy(d0_hbm.at[wid], d0_v)
    pltpu.sync_copy(d1_hbm.at[wid], d1_v)

    zrow = jnp.zeros((16,), jnp.float32)

    def zloop(r, carry):
        for j in range(D // 16):
            z_v[r, pl.ds(j * 16, 16)] = zrow
        return carry

    lax.fori_loop(0, 4, zloop, 0)
    pltpu.sync_copy(z_v, disp_hbm.at[pl.ds(NSLOT + wid * 4, 4)])

    bufs = (xb0_v, xb1_v)
    sems = (sem0, sem1)
    nch = TOK_W // DCH

    def start(c):
        return pltpu.async_copy(
            x_hbm.at[pl.ds(tbase + c * DCH, DCH)], bufs[c % 2], sems[c % 2])

    cps = {0: start(0)}
    for c in range(nch):
        cps.pop(c).wait()
        if c + 1 < nch:
            cps[c + 1] = start(c + 1)
        pltpu.sync_copy(bufs[c % 2], disp_hbm.at[d0_v.at[c]])
        pltpu.sync_copy(bufs[c % 2], disp_hbm.at[d1_v.at[c]])


def _combine_body(eo_hbm, ci0_hbm, ci1_hbm, w0_hbm, w1_hbm, y_hbm,
                  i0_v, i1_v, w0_v, w1_v, a0_v, b0_v, a1_v, b1_v, sem0, sem1):
    wid = lax.axis_index("s") * 2 + lax.axis_index("c")
    tbase = wid * TOK_W
    pltpu.sync_copy(ci0_hbm.at[pl.ds(tbase, TOK_W)], i0_v)
    pltpu.sync_copy(ci1_hbm.at[pl.ds(tbase, TOK_W)], i1_v)
    pltpu.sync_copy(w0_hbm.at[pl.ds(tbase, TOK_W)], w0_v)
    pltpu.sync_copy(w1_hbm.at[pl.ds(tbase, TOK_W)], w1_v)
    abufs = (a0_v, a1_v)
    bbufs = (b0_v, b1_v)
    sems = (sem0, sem1)
    nch = TOK_W // CCH

    def start(c):
        s = c % 2
        idx0 = i0_v[pl.ds(c * CCH, CCH)]
        idx1 = i1_v[pl.ds(c * CCH, CCH)]
        cpa = pltpu.async_copy(eo_hbm.at[idx0], abufs[s], sems[s])
        cpb = pltpu.async_copy(eo_hbm.at[idx1], bbufs[s], sems[s])
        return cpa, cpb

    cps = {0: start(0)}
    for c in range(nch):
        s = c % 2
        cpa, cpb = cps.pop(c)
        cpa.wait()
        cpb.wait()
        if c + 1 < nch:
            cps[c + 1] = start(c + 1)
        a_v = abufs[s]
        b_v = bbufs[s]
        wv0 = w0_v[pl.ds(c * CCH, CCH)]
        wv1 = w1_v[pl.ds(c * CCH, CCH)]

        def jadd(j, carry):
            sl = pl.ds(j * 16, 16)
            for r in range(CCH):
                a_v[r, sl] = a_v[r, sl] * wv0[r] + b_v[r, sl] * wv1[r]
            return carry

        lax.fori_loop(0, D // 16, jadd, 0)
        pltpu.sync_copy(a_v, y_hbm.at[pl.ds(tbase + c * CCH, CCH)])


@functools.lru_cache(maxsize=None)
def _sc_kernels():
    mesh = plsc.VectorSubcoreMesh(core_axis_name="c", subcore_axis_name="s")

    dispatch = pl.kernel(
        _dispatch_body,
        out_type=jax.ShapeDtypeStruct((DISP_ROWS, D), jnp.float32),
        mesh=mesh,
        compiler_params=pltpu.CompilerParams(needs_layout_passes=False),
        scratch_types=[
            pltpu.VMEM((4, DCH), jnp.int32),
            pltpu.VMEM((4, DCH), jnp.int32),
            pltpu.VMEM((DCH, D), jnp.float32),
            pltpu.VMEM((DCH, D), jnp.float32),
            pltpu.VMEM((4, D), jnp.float32),
            pltpu.SemaphoreType.DMA,
            pltpu.SemaphoreType.DMA,
        ],
    )

    combine = pl.kernel(
        _combine_body,
        out_type=jax.ShapeDtypeStruct((T, D), jnp.float32),
        mesh=mesh,
        compiler_params=pltpu.CompilerParams(needs_layout_passes=False),
        scratch_types=[
            pltpu.VMEM((TOK_W,), jnp.int32),
            pltpu.VMEM((TOK_W,), jnp.int32),
            pltpu.VMEM((TOK_W,), jnp.float32),
            pltpu.VMEM((TOK_W,), jnp.float32),
            pltpu.VMEM((CCH, D), jnp.float32),
            pltpu.VMEM((CCH, D), jnp.float32),
            pltpu.VMEM((CCH, D), jnp.float32),
            pltpu.VMEM((CCH, D), jnp.float32),
            pltpu.SemaphoreType.DMA,
            pltpu.SemaphoreType.DMA,
        ],
    )
    return dispatch, combine


def _bmm_body(disp_ref, wg_ref, wu_ref, wd_ref, out_ref):
    xb = disp_ref[...]
    g = jnp.dot(xb, wg_ref[0], preferred_element_type=jnp.float32)
    u = jnp.dot(xb, wu_ref[0], preferred_element_type=jnp.float32)
    h = g * jax.nn.sigmoid(g) * u
    out_ref[...] = jnp.dot(h, wd_ref[0], preferred_element_type=jnp.float32)


def _bmm(disp, w_gate, w_up, w_down, interpret=False):
    nblk = NROW // 128
    eidx = lambda i: (jnp.minimum(i // 8, E - 1), 0, 0)
    return pl.pallas_call(
        _bmm_body,
        grid=(nblk,),
        in_specs=[
            pl.BlockSpec((128, D), lambda i: (i, 0)),
            pl.BlockSpec((1, D, I), eidx),
            pl.BlockSpec((1, D, I), eidx),
            pl.BlockSpec((1, I, D), eidx),
        ],
        out_specs=pl.BlockSpec((128, D), lambda i: (i, 0)),
        out_shape=jax.ShapeDtypeStruct((NROW, D), jnp.float32),
        interpret=interpret,
    )(disp, w_gate, w_up, w_down)


@jax.jit
def kernel(x, wg, w_gate, w_up, w_down):
    dispatch, combine = _sc_kernels()
    d0, d1, ci0, ci1, wk0, wk1, laux = _router(x, wg)
    disp = dispatch(x, d0, d1)
    eo = _bmm(disp, w_gate, w_up, w_down)
    y = combine(eo, ci0.reshape(T), ci1.reshape(T),
                wk0.reshape(T), wk1.reshape(T))
    return y, laux.reshape(())


# speedup vs baseline: 1.0275x; 1.0275x over previous
import functools

import jax
import jax.numpy as jnp
from jax import lax
from jax.experimental import pallas as pl
from jax.experimental.pallas import tpu as pltpu
from jax.experimental.pallas import tpu_sc as plsc

T = 4096
D = 1024
E = 8
I = 512
CAP = 1024
NSLOT = E * CAP
ZROW = NSLOT
NROW = 8320
DUMP = NROW
DISP_ROWS = 8448
NB = 8
BT = T // NB
NW = 32
TOK_W = T // NW
DCH = 32
CCH = 16


def _router_body(x_ref, wg_ref, d0_ref, d1_ref, ci0_ref, ci1_ref,
                 wk0_ref, wk1_ref, laux_ref,
                 ti0_s, ti1_s, p0_s, p1_s, w0_s, w1_s, off0, off1, me_acc):
    p = pl.program_id(0)
    b = pl.program_id(1)

    @pl.when(jnp.logical_and(p == 0, b == 0))
    def _init():
        off0[...] = jnp.zeros_like(off0)
        off1[...] = jnp.zeros_like(off1)
        me_acc[...] = jnp.zeros_like(me_acc)

    tok_sl = pl.ds(b * BT, BT)

    @pl.when(p == 0)
    def _pass0():
        xb = x_ref[...]
        wg = wg_ref[...]
        logits = jnp.dot(xb, wg, preferred_element_type=jnp.float32)
        m = jnp.max(logits, axis=1, keepdims=True)
        ex = jnp.exp(logits - m)
        gates = ex / jnp.sum(ex, axis=1, keepdims=True)

        lane = lax.broadcasted_iota(jnp.int32, (BT, E), 1)
        v0 = jnp.max(gates, axis=1, keepdims=True)
        i0 = jnp.min(jnp.where(gates == v0, lane, E), axis=1, keepdims=True)
        g1 = jnp.where(lane == i0, -jnp.inf, gates)
        v1 = jnp.max(g1, axis=1, keepdims=True)
        i1 = jnp.min(jnp.where(g1 == v1, lane, E), axis=1, keepdims=True)
        denom = v0 + v1 + 1e-9
        mask0 = (lane == i0).astype(jnp.float32)
        mask1 = (lane == i1).astype(jnp.float32)

        tri = (lax.broadcasted_iota(jnp.int32, (BT, BT), 0)
               >= lax.broadcasted_iota(jnp.int32, (BT, BT), 1)
               ).astype(jnp.float32)
        c0 = jnp.dot(tri, mask0, preferred_element_type=jnp.float32)
        c1 = jnp.dot(tri, mask1, preferred_element_type=jnp.float32)

        o0 = off0[...]
        o1 = off1[...]
        pos0 = jnp.sum(mask0 * (c0 - 1.0 + o0), axis=1, keepdims=True)
        pos1 = jnp.sum(mask1 * (c1 - 1.0 + o1), axis=1, keepdims=True)
        off0[...] = o0 + jnp.sum(mask0, axis=0, keepdims=True)
        off1[...] = o1 + jnp.sum(mask1, axis=0, keepdims=True)
        me_acc[...] = me_acc[...] + jnp.sum(gates, axis=0, keepdims=True)

        ti0_s[tok_sl, :] = i0
        ti1_s[tok_sl, :] = i1
        p0_s[tok_sl, :] = pos0.astype(jnp.int32)
        p1_s[tok_sl, :] = pos1.astype(jnp.int32)
        w0_s[tok_sl, :] = v0 / denom
        w1_s[tok_sl, :] = v1 / denom

    @pl.when(p == 1)
    def _pass1():
        i0 = ti0_s[tok_sl, :]
        i1 = ti1_s[tok_sl, :]
        pos0 = p0_s[tok_sl, :]
        w0 = w0_s[tok_sl, :]
        w1 = w1_s[tok_sl, :]
        lane = lax.broadcasted_iota(jnp.int32, (BT, E), 1)
        mask1 = (lane == i1).astype(jnp.float32)
        cnt_g = jnp.sum(mask1 * off0[...], axis=1, keepdims=True)
        pos1 = p1_s[tok_sl, :] + cnt_g.astype(jnp.int32)

        k0 = pos0 < CAP
        k1 = pos1 < CAP
        s0 = i0 * CAP + pos0
        s1 = i1 * CAP + pos1
        d0 = jnp.where(k0, s0, DUMP)
        d1 = jnp.where(k1, s1, DUMP)
        d0_ref[...] = d0.reshape(4, 4, DCH)
        d1_ref[...] = d1.reshape(4, 4, DCH)
        ci0_ref[...] = jnp.where(k0, s0, ZROW).reshape(1, BT, 1)
        ci1_ref[...] = jnp.where(k1, s1, ZROW).reshape(1, BT, 1)
        wk0_ref[...] = jnp.where(k0, w0, 0.0).reshape(1, BT, 1)
        wk1_ref[...] = jnp.where(k1, w1, 0.0).reshape(1, BT, 1)

        @pl.when(b == NB - 1)
        def _fin():
            me = me_acc[...] / float(T)
            ce = off0[...] / float(T)
            laux_ref[...] = jnp.sum(me * ce).reshape(1, 1) * float(E)


def _router(x, wg):
    out_shapes = (
        jax.ShapeDtypeStruct((NW, 4, DCH), jnp.int32),
        jax.ShapeDtypeStruct((NW, 4, DCH), jnp.int32),
        jax.ShapeDtypeStruct((NB, BT, 1), jnp.int32),
        jax.ShapeDtypeStruct((NB, BT, 1), jnp.int32),
        jax.ShapeDtypeStruct((NB, BT, 1), jnp.float32),
        jax.ShapeDtypeStruct((NB, BT, 1), jnp.float32),
        jax.ShapeDtypeStruct((1, 1), jnp.float32),
    )
    blk = pl.BlockSpec((1, BT, 1), lambda p, b: (b, 0, 0))
    dblk = pl.BlockSpec((4, 4, DCH), lambda p, b: (b, 0, 0))
    return pl.pallas_call(
        _router_body,
        grid=(2, NB),
        in_specs=[
            pl.BlockSpec((BT, D), lambda p, b: (b * (1 - p), 0)),
            pl.BlockSpec((D, E), lambda p, b: (0, 0)),
        ],
        out_specs=(dblk, dblk, blk, blk, blk, blk,
                   pl.BlockSpec((1, 1), lambda p, b: (0, 0))),
        out_shape=out_shapes,
        scratch_shapes=[
            pltpu.VMEM((T, 1), jnp.int32),
            pltpu.VMEM((T, 1), jnp.int32),
            pltpu.VMEM((T, 1), jnp.int32),
            pltpu.VMEM((T, 1), jnp.int32),
            pltpu.VMEM((T, 1), jnp.float32),
            pltpu.VMEM((T, 1), jnp.float32),
            pltpu.VMEM((1, E), jnp.float32),
            pltpu.VMEM((1, E), jnp.float32),
            pltpu.VMEM((1, E), jnp.float32),
        ],
    )(x, wg)


def _dispatch_body(x_hbm, d0_hbm, d1_hbm, disp_hbm,
                   d0_v, d1_v, xb0_v, xb1_v, z_v, sem0, sem1, ssem0, ssem1):
    wid = lax.axis_index("s") * 2 + lax.axis_index("c")
    tbase = wid * TOK_W

    pltpu.sync_copy(d0_hbm.at[wid], d0_v)
    pltpu.sync_copy(d1_hbm.at[wid], d1_v)

    zrow = jnp.zeros((16,), jnp.float32)

    def zloop(r, carry):
        for j in range(D // 16):
            z_v[r, pl.ds(j * 16, 16)] = zrow
        return carry

    lax.fori_loop(0, 4, zloop, 0)
    pltpu.sync_copy(z_v, disp_hbm.at[pl.ds(NSLOT + wid * 4, 4)])

    bufs = (xb0_v, xb1_v)
    sems = (sem0, sem1)
    nch = TOK_W // DCH

    def start(c):
        return pltpu.async_copy(
            x_hbm.at[pl.ds(tbase + c * DCH, DCH)], bufs[c % 2], sems[c % 2])

    ssems = (ssem0, ssem1)
    cps = {0: start(0)}
    scps = {}
    for c in range(nch):
        if c + 1 < nch:
            if c - 1 in scps:
                for cp in scps.pop(c - 1):
                    cp.wait()
            cps[c + 1] = start(c + 1)
        cps.pop(c).wait()
        scps[c] = (
            pltpu.async_copy(bufs[c % 2], disp_hbm.at[d0_v.at[c]],
                             ssems[c % 2]),
            pltpu.async_copy(bufs[c % 2], disp_hbm.at[d1_v.at[c]],
                             ssems[c % 2]),
        )
    for c in sorted(scps):
        for cp in scps.pop(c):
            cp.wait()


def _combine_body(eo_hbm, ci0_hbm, ci1_hbm, w0_hbm, w1_hbm, y_hbm,
                  i0_v, i1_v, w0_v, w1_v, a0_v, b0_v, a1_v, b1_v,
                  sem0, sem1, ssem0, ssem1):
    wid = lax.axis_index("s") * 2 + lax.axis_index("c")
    tbase = wid * TOK_W
    pltpu.sync_copy(ci0_hbm.at[pl.ds(tbase, TOK_W)], i0_v)
    pltpu.sync_copy(ci1_hbm.at[pl.ds(tbase, TOK_W)], i1_v)
    pltpu.sync_copy(w0_hbm.at[pl.ds(tbase, TOK_W)], w0_v)
    pltpu.sync_copy(w1_hbm.at[pl.ds(tbase, TOK_W)], w1_v)
    abufs = (a0_v, a1_v)
    bbufs = (b0_v, b1_v)
    sems = (sem0, sem1)
    nch = TOK_W // CCH

    def start(c):
        s = c % 2
        idx0 = i0_v[pl.ds(c * CCH, CCH)]
        idx1 = i1_v[pl.ds(c * CCH, CCH)]
        cpa = pltpu.async_copy(eo_hbm.at[idx0], abufs[s], sems[s])
        cpb = pltpu.async_copy(eo_hbm.at[idx1], bbufs[s], sems[s])
        return cpa, cpb

    ssems = (ssem0, ssem1)
    cps = {0: start(0)}
    scps = {}
    for c in range(nch):
        s = c % 2
        if c + 1 < nch:
            if c - 1 in scps:
                scps.pop(c - 1).wait()
            cps[c + 1] = start(c + 1)
        cpa, cpb = cps.pop(c)
        cpa.wait()
        cpb.wait()
        a_v = abufs[s]
        b_v = bbufs[s]
        wv0 = w0_v[pl.ds(c * CCH, CCH)]
        wv1 = w1_v[pl.ds(c * CCH, CCH)]

        def jadd(j, carry):
            sl = pl.ds(j * 16, 16)
            for r in range(CCH):
                a_v[r, sl] = a_v[r, sl] * wv0[r] + b_v[r, sl] * wv1[r]
            return carry

        lax.fori_loop(0, D // 16, jadd, 0)
        scps[c] = pltpu.async_copy(
            a_v, y_hbm.at[pl.ds(tbase + c * CCH, CCH)], ssems[s])
    for c in sorted(scps):
        scps.pop(c).wait()


@functools.lru_cache(maxsize=None)
def _sc_kernels():
    mesh = plsc.VectorSubcoreMesh(core_axis_name="c", subcore_axis_name="s")

    dispatch = pl.kernel(
        _dispatch_body,
        out_type=jax.ShapeDtypeStruct((DISP_ROWS, D), jnp.float32),
        mesh=mesh,
        compiler_params=pltpu.CompilerParams(needs_layout_passes=False),
        scratch_types=[
            pltpu.VMEM((4, DCH), jnp.int32),
            pltpu.VMEM((4, DCH), jnp.int32),
            pltpu.VMEM((DCH, D), jnp.float32),
            pltpu.VMEM((DCH, D), jnp.float32),
            pltpu.VMEM((4, D), jnp.float32),
            pltpu.SemaphoreType.DMA,
            pltpu.SemaphoreType.DMA,
            pltpu.SemaphoreType.DMA,
            pltpu.SemaphoreType.DMA,
        ],
    )

    combine = pl.kernel(
        _combine_body,
        out_type=jax.ShapeDtypeStruct((T, D), jnp.float32),
        mesh=mesh,
        compiler_params=pltpu.CompilerParams(needs_layout_passes=False),
        scratch_types=[
            pltpu.VMEM((TOK_W,), jnp.int32),
            pltpu.VMEM((TOK_W,), jnp.int32),
            pltpu.VMEM((TOK_W,), jnp.float32),
            pltpu.VMEM((TOK_W,), jnp.float32),
            pltpu.VMEM((CCH, D), jnp.float32),
            pltpu.VMEM((CCH, D), jnp.float32),
            pltpu.VMEM((CCH, D), jnp.float32),
            pltpu.VMEM((CCH, D), jnp.float32),
            pltpu.SemaphoreType.DMA,
            pltpu.SemaphoreType.DMA,
            pltpu.SemaphoreType.DMA,
            pltpu.SemaphoreType.DMA,
        ],
    )
    return dispatch, combine


def _bmm_body(disp_ref, wg_ref, wu_ref, wd_ref, out_ref):
    xb = disp_ref[...]
    g = jnp.dot(xb, wg_ref[0], preferred_element_type=jnp.float32)
    u = jnp.dot(xb, wu_ref[0], preferred_element_type=jnp.float32)
    h = g * jax.nn.sigmoid(g) * u
    out_ref[...] = jnp.dot(h, wd_ref[0], preferred_element_type=jnp.float32)


def _bmm(disp, w_gate, w_up, w_down, interpret=False):
    nblk = NROW // 128
    eidx = lambda i: (jnp.minimum(i // 8, E - 1), 0, 0)
    return pl.pallas_call(
        _bmm_body,
        grid=(nblk,),
        in_specs=[
            pl.BlockSpec((128, D), lambda i: (i, 0)),
            pl.BlockSpec((1, D, I), eidx),
            pl.BlockSpec((1, D, I), eidx),
            pl.BlockSpec((1, I, D), eidx),
        ],
        out_specs=pl.BlockSpec((128, D), lambda i: (i, 0)),
        out_shape=jax.ShapeDtypeStruct((NROW, D), jnp.float32),
        interpret=interpret,
    )(disp, w_gate, w_up, w_down)


@jax.jit
def kernel(x, wg, w_gate, w_up, w_down):
    dispatch, combine = _sc_kernels()
    d0, d1, ci0, ci1, wk0, wk1, laux = _router(x, wg)
    disp = dispatch(x, d0, d1)
    eo = _bmm(disp, w_gate, w_up, w_down)
    y = combine(eo, ci0.reshape(T), ci1.reshape(T),
                wk0.reshape(T), wk1.reshape(T))
    return y, laux.reshape(())


# speedup vs baseline: 1.1502x; 1.1194x over previous
import functools

import jax
import jax.numpy as jnp
from jax import lax
from jax.experimental import pallas as pl
from jax.experimental.pallas import tpu as pltpu
from jax.experimental.pallas import tpu_sc as plsc

T = 4096
D = 1024
E = 8
I = 512
CAP = 1024
NSLOT = E * CAP
ZROW = NSLOT
NROW = 8448
DUMP = NROW
DISP_ROWS = 8704
NB = 8
BT = T // NB
NW = 32
TOK_W = T // NW
DCH = 32
CCH = 16


def _router_body(x_ref, wg_ref, d0_ref, d1_ref, ci0_ref, ci1_ref,
                 wk0_ref, wk1_ref, laux_ref,
                 ti0_s, ti1_s, p0_s, p1_s, w0_s, w1_s, off0, off1, me_acc):
    p = pl.program_id(0)
    b = pl.program_id(1)

    @pl.when(jnp.logical_and(p == 0, b == 0))
    def _init():
        off0[...] = jnp.zeros_like(off0)
        off1[...] = jnp.zeros_like(off1)
        me_acc[...] = jnp.zeros_like(me_acc)

    tok_sl = pl.ds(b * BT, BT)

    @pl.when(p == 0)
    def _pass0():
        xb = x_ref[...]
        wg = wg_ref[...]
        logits = jnp.dot(xb, wg, preferred_element_type=jnp.float32)
        m = jnp.max(logits, axis=1, keepdims=True)
        ex = jnp.exp(logits - m)
        gates = ex / jnp.sum(ex, axis=1, keepdims=True)

        lane = lax.broadcasted_iota(jnp.int32, (BT, E), 1)
        v0 = jnp.max(gates, axis=1, keepdims=True)
        i0 = jnp.min(jnp.where(gates == v0, lane, E), axis=1, keepdims=True)
        g1 = jnp.where(lane == i0, -jnp.inf, gates)
        v1 = jnp.max(g1, axis=1, keepdims=True)
        i1 = jnp.min(jnp.where(g1 == v1, lane, E), axis=1, keepdims=True)
        denom = v0 + v1 + 1e-9
        mask0 = (lane == i0).astype(jnp.float32)
        mask1 = (lane == i1).astype(jnp.float32)

        tri = (lax.broadcasted_iota(jnp.int32, (BT, BT), 0)
               >= lax.broadcasted_iota(jnp.int32, (BT, BT), 1)
               ).astype(jnp.float32)
        c0 = jnp.dot(tri, mask0, preferred_element_type=jnp.float32)
        c1 = jnp.dot(tri, mask1, preferred_element_type=jnp.float32)

        o0 = off0[...]
        o1 = off1[...]
        pos0 = jnp.sum(mask0 * (c0 - 1.0 + o0), axis=1, keepdims=True)
        pos1 = jnp.sum(mask1 * (c1 - 1.0 + o1), axis=1, keepdims=True)
        off0[...] = o0 + jnp.sum(mask0, axis=0, keepdims=True)
        off1[...] = o1 + jnp.sum(mask1, axis=0, keepdims=True)
        me_acc[...] = me_acc[...] + jnp.sum(gates, axis=0, keepdims=True)

        ti0_s[tok_sl, :] = i0
        ti1_s[tok_sl, :] = i1
        p0_s[tok_sl, :] = pos0.astype(jnp.int32)
        p1_s[tok_sl, :] = pos1.astype(jnp.int32)
        w0_s[tok_sl, :] = v0 / denom
        w1_s[tok_sl, :] = v1 / denom

    @pl.when(p == 1)
    def _pass1():
        i0 = ti0_s[tok_sl, :]
        i1 = ti1_s[tok_sl, :]
        pos0 = p0_s[tok_sl, :]
        w0 = w0_s[tok_sl, :]
        w1 = w1_s[tok_sl, :]
        lane = lax.broadcasted_iota(jnp.int32, (BT, E), 1)
        mask1 = (lane == i1).astype(jnp.float32)
        cnt_g = jnp.sum(mask1 * off0[...], axis=1, keepdims=True)
        pos1 = p1_s[tok_sl, :] + cnt_g.astype(jnp.int32)

        k0 = pos0 < CAP
        k1 = pos1 < CAP
        s0 = i0 * CAP + pos0
        s1 = i1 * CAP + pos1
        d0 = jnp.where(k0, s0, DUMP)
        d1 = jnp.where(k1, s1, DUMP)
        d0_ref[...] = d0.reshape(4, 4, DCH)
        d1_ref[...] = d1.reshape(4, 4, DCH)
        ci0_ref[...] = jnp.where(k0, s0, ZROW).reshape(1, BT, 1)
        ci1_ref[...] = jnp.where(k1, s1, ZROW).reshape(1, BT, 1)
        wk0_ref[...] = jnp.where(k0, w0, 0.0).reshape(1, BT, 1)
        wk1_ref[...] = jnp.where(k1, w1, 0.0).reshape(1, BT, 1)

        @pl.when(b == NB - 1)
        def _fin():
            me = me_acc[...] / float(T)
            ce = off0[...] / float(T)
            laux_ref[...] = jnp.sum(me * ce).reshape(1, 1) * float(E)


def _router(x, wg):
    out_shapes = (
        jax.ShapeDtypeStruct((NW, 4, DCH), jnp.int32),
        jax.ShapeDtypeStruct((NW, 4, DCH), jnp.int32),
        jax.ShapeDtypeStruct((NB, BT, 1), jnp.int32),
        jax.ShapeDtypeStruct((NB, BT, 1), jnp.int32),
        jax.ShapeDtypeStruct((NB, BT, 1), jnp.float32),
        jax.ShapeDtypeStruct((NB, BT, 1), jnp.float32),
        jax.ShapeDtypeStruct((1, 1), jnp.float32),
    )
    blk = pl.BlockSpec((1, BT, 1), lambda p, b: (b, 0, 0))
    dblk = pl.BlockSpec((4, 4, DCH), lambda p, b: (b, 0, 0))
    return pl.pallas_call(
        _router_body,
        grid=(2, NB),
        in_specs=[
            pl.BlockSpec((BT, D), lambda p, b: (b * (1 - p), 0)),
            pl.BlockSpec((D, E), lambda p, b: (0, 0)),
        ],
        out_specs=(dblk, dblk, blk, blk, blk, blk,
                   pl.BlockSpec((1, 1), lambda p, b: (0, 0))),
        out_shape=out_shapes,
        scratch_shapes=[
            pltpu.VMEM((T, 1), jnp.int32),
            pltpu.VMEM((T, 1), jnp.int32),
            pltpu.VMEM((T, 1), jnp.int32),
            pltpu.VMEM((T, 1), jnp.int32),
            pltpu.VMEM((T, 1), jnp.float32),
            pltpu.VMEM((T, 1), jnp.float32),
            pltpu.VMEM((1, E), jnp.float32),
            pltpu.VMEM((1, E), jnp.float32),
            pltpu.VMEM((1, E), jnp.float32),
        ],
    )(x, wg)


def _dispatch_body(x_hbm, d0_hbm, d1_hbm, disp_hbm,
                   d0_v, d1_v, xb0_v, xb1_v, z_v, sem0, sem1):
    wid = lax.axis_index("s") * 2 + lax.axis_index("c")
    tbase = wid * TOK_W

    pltpu.sync_copy(d0_hbm.at[wid], d0_v)
    pltpu.sync_copy(d1_hbm.at[wid], d1_v)

    zrow = jnp.zeros((16,), jnp.float32)

    def zloop(r, carry):
        for j in range(D // 16):
            z_v[r, pl.ds(j * 16, 16)] = zrow
        return carry

    lax.fori_loop(0, 8, zloop, 0)
    pltpu.sync_copy(z_v, disp_hbm.at[pl.ds(NSLOT + wid * 8, 8)])

    bufs = (xb0_v, xb1_v)
    sems = (sem0, sem1)
    nch = TOK_W // DCH

    def start(c):
        return pltpu.async_copy(
            x_hbm.at[pl.ds(tbase + c * DCH, DCH)], bufs[c % 2], sems[c % 2])

    cps = {0: start(0)}
    for c in range(nch):
        cps.pop(c).wait()
        if c + 1 < nch:
            cps[c + 1] = start(c + 1)
        pltpu.sync_copy(bufs[c % 2], disp_hbm.at[d0_v.at[c]])
        pltpu.sync_copy(bufs[c % 2], disp_hbm.at[d1_v.at[c]])


def _combine_body(eo_hbm, ci0_hbm, ci1_hbm, w0_hbm, w1_hbm, y_hbm,
                  i0_v, i1_v, w0_v, w1_v, a0_v, b0_v, a1_v, b1_v, sem0, sem1):
    wid = lax.axis_index("s") * 2 + lax.axis_index("c")
    tbase = wid * TOK_W
    pltpu.sync_copy(ci0_hbm.at[pl.ds(tbase, TOK_W)], i0_v)
    pltpu.sync_copy(ci1_hbm.at[pl.ds(tbase, TOK_W)], i1_v)
    pltpu.sync_copy(w0_hbm.at[pl.ds(tbase, TOK_W)], w0_v)
    pltpu.sync_copy(w1_hbm.at[pl.ds(tbase, TOK_W)], w1_v)
    abufs = (a0_v, a1_v)
    bbufs = (b0_v, b1_v)
    sems = (sem0, sem1)
    nch = TOK_W // CCH

    def start(c):
        s = c % 2
        idx0 = i0_v[pl.ds(c * CCH, CCH)]
        idx1 = i1_v[pl.ds(c * CCH, CCH)]
        cpa = pltpu.async_copy(eo_hbm.at[idx0], abufs[s], sems[s])
        cpb = pltpu.async_copy(eo_hbm.at[idx1], bbufs[s], sems[s])
        return cpa, cpb

    cps = {0: start(0)}
    for c in range(nch):
        s = c % 2
        cpa, cpb = cps.pop(c)
        cpa.wait()
        cpb.wait()
        if c + 1 < nch:
            cps[c + 1] = start(c + 1)
        a_v = abufs[s]
        b_v = bbufs[s]
        wv0 = w0_v[pl.ds(c * CCH, CCH)]
        wv1 = w1_v[pl.ds(c * CCH, CCH)]

        def jadd(j, carry):
            sl = pl.ds(j * 16, 16)
            for r in range(CCH):
                a_v[r, sl] = a_v[r, sl] * wv0[r] + b_v[r, sl] * wv1[r]
            return carry

        lax.fori_loop(0, D // 16, jadd, 0)
        pltpu.sync_copy(a_v, y_hbm.at[pl.ds(tbase + c * CCH, CCH)])


@functools.lru_cache(maxsize=None)
def _sc_kernels():
    mesh = plsc.VectorSubcoreMesh(core_axis_name="c", subcore_axis_name="s")

    dispatch = pl.kernel(
        _dispatch_body,
        out_type=jax.ShapeDtypeStruct((DISP_ROWS, D), jnp.float32),
        mesh=mesh,
        compiler_params=pltpu.CompilerParams(needs_layout_passes=False),
        scratch_types=[
            pltpu.VMEM((4, DCH), jnp.int32),
            pltpu.VMEM((4, DCH), jnp.int32),
            pltpu.VMEM((DCH, D), jnp.float32),
            pltpu.VMEM((DCH, D), jnp.float32),
            pltpu.VMEM((8, D), jnp.float32),
            pltpu.SemaphoreType.DMA,
            pltpu.SemaphoreType.DMA,
        ],
    )

    combine = pl.kernel(
        _combine_body,
        out_type=jax.ShapeDtypeStruct((T, D), jnp.float32),
        mesh=mesh,
        compiler_params=pltpu.CompilerParams(needs_layout_passes=False),
        scratch_types=[
            pltpu.VMEM((TOK_W,), jnp.int32),
            pltpu.VMEM((TOK_W,), jnp.int32),
            pltpu.VMEM((TOK_W,), jnp.float32),
            pltpu.VMEM((TOK_W,), jnp.float32),
            pltpu.VMEM((CCH, D), jnp.float32),
            pltpu.VMEM((CCH, D), jnp.float32),
            pltpu.VMEM((CCH, D), jnp.float32),
            pltpu.VMEM((CCH, D), jnp.float32),
            pltpu.SemaphoreType.DMA,
            pltpu.SemaphoreType.DMA,
        ],
    )
    return dispatch, combine


def _bmm_body(disp_ref, wg_ref, wu_ref, wd_ref, out_ref):
    xb = disp_ref[...]
    g = jnp.dot(xb, wg_ref[0], preferred_element_type=jnp.float32)
    u = jnp.dot(xb, wu_ref[0], preferred_element_type=jnp.float32)
    h = g * jax.nn.sigmoid(g) * u
    out_ref[...] = jnp.dot(h, wd_ref[0], preferred_element_type=jnp.float32)


def _bmm(disp, w_gate, w_up, w_down, interpret=False):
    nblk = NROW // 256
    eidx = lambda i: (jnp.minimum(i // 4, E - 1), 0, 0)
    return pl.pallas_call(
        _bmm_body,
        grid=(nblk,),
        in_specs=[
            pl.BlockSpec((256, D), lambda i: (i, 0)),
            pl.BlockSpec((1, D, I), eidx),
            pl.BlockSpec((1, D, I), eidx),
            pl.BlockSpec((1, I, D), eidx),
        ],
        out_specs=pl.BlockSpec((256, D), lambda i: (i, 0)),
        out_shape=jax.ShapeDtypeStruct((NROW, D), jnp.float32),
        interpret=interpret,
    )(disp, w_gate, w_up, w_down)


@jax.jit
def kernel(x, wg, w_gate, w_up, w_down):
    dispatch, combine = _sc_kernels()
    d0, d1, ci0, ci1, wk0, wk1, laux = _router(x, wg)
    disp = dispatch(x, d0, d1)
    eo = _bmm(disp, w_gate, w_up, w_down)
    y = combine(eo, ci0.reshape(T), ci1.reshape(T),
                wk0.reshape(T), wk1.reshape(T))
    return y, laux.reshape(())
